# Initial kernel scaffold; baseline (speedup 1.0000x reference)
import jax
import jax.numpy as jnp
from jax.experimental import pallas as pl


def kernel(x, anatomical_ids, phase_ids, pe, anat_table, phase_table):
    raise NotImplementedError("write your pallas kernel here")


# speedup vs baseline: 1.8186x; 1.8186x over previous
import jax
import jax.numpy as jnp
from jax.experimental import pallas as pl

_SEQ_BLK = 512


def _pe_body(x_ref, pe_ref, aid_ref, pid_ref, anat_ref, phase_ref, out_ref):
    x = x_ref[...]
    pe = pe_ref[...]
    aid = aid_ref[0]
    pid = pid_ref[0]

    n_anat = anat_ref.shape[0]
    n_phase = phase_ref.shape[0]
    sb = x.shape[0]

    anat_t = jnp.concatenate([anat_ref[...]] * 4, axis=1)
    phase_t = jnp.concatenate([phase_ref[...]] * 4, axis=1)

    a_lane = jax.lax.broadcasted_iota(jnp.int32, (sb, n_anat), 1)
    p_lane = jax.lax.broadcasted_iota(jnp.int32, (sb, n_phase), 1)
    oh_a = (aid == a_lane).astype(jnp.float32)
    oh_p = (pid == p_lane).astype(jnp.float32)

    enc = jax.lax.dot(oh_a, anat_t, precision=jax.lax.Precision.HIGHEST)
    enc = enc + jax.lax.dot(oh_p, phase_t, precision=jax.lax.Precision.HIGHEST)
    out_ref[...] = x + pe + enc


def kernel(x, anatomical_ids, phase_ids, pe, anat_table, phase_table):
    seq_len, batch, d_model = x.shape
    sblk = min(_SEQ_BLK, seq_len)
    n_sblk = seq_len // sblk

    x2 = x.reshape(seq_len, batch * d_model)
    pe2 = pe[:seq_len, 0, :]
    aid = anatomical_ids.astype(jnp.int32).T.reshape(batch, seq_len, 1)
    pid = phase_ids.astype(jnp.int32).T.reshape(batch, seq_len, 1)

    out2 = pl.pallas_call(
        _pe_body,
        grid=(n_sblk, batch),
        in_specs=[
            pl.BlockSpec((sblk, d_model), lambda i, b: (i, b)),
            pl.BlockSpec((sblk, d_model), lambda i, b: (i, 0)),
            pl.BlockSpec((1, sblk, 1), lambda i, b: (b, i, 0)),
            pl.BlockSpec((1, sblk, 1), lambda i, b: (b, i, 0)),
            pl.BlockSpec(anat_table.shape, lambda i, b: (0, 0)),
            pl.BlockSpec(phase_table.shape, lambda i, b: (0, 0)),
        ],
        out_specs=pl.BlockSpec((sblk, d_model), lambda i, b: (i, b)),
        out_shape=jax.ShapeDtypeStruct((seq_len, batch * d_model), x.dtype),
    )(x2, pe2, aid, pid, anat_table, phase_table)
    return out2.reshape(seq_len, batch, d_model)
